# baseline (speedup 1.0000x reference)
import functools

import jax
import jax.numpy as jnp
from jax import lax
from jax.experimental import pallas as pl
from jax.experimental.pallas import tpu as pltpu
from jax.experimental.pallas import tpu_sc as plsc

_DIM = 64
_PAD = 128
_NW = 32
_TOK = 128
_NBUF = 4


@functools.lru_cache(maxsize=None)
def _make_gather(n_rows: int, n_cols: int):
    mesh = plsc.VectorSubcoreMesh(core_axis_name="c", subcore_axis_name="s")

    @functools.partial(
        pl.kernel,
        out_type=jax.ShapeDtypeStruct((n_cols, _DIM, n_rows), jnp.float32),
        mesh=mesh,
        compiler_params=pltpu.CompilerParams(
            use_tc_tiling_on_sc=True, needs_layout_passes=False
        ),
        scratch_types=[pltpu.VMEM((n_cols, _TOK), jnp.int32)]
        + [pltpu.VMEM((_TOK, _PAD), jnp.float32)] * _NBUF
        + [pltpu.VMEM((1, _DIM, _TOK), jnp.float32)] * _NBUF
        + [pltpu.SemaphoreType.DMA] * (2 * _NBUF),
    )
    def k(idx_hbm, table_hbm, out_hbm, idx_v, *rest):
        g_buf = rest[:_NBUF]
        t_buf = rest[_NBUF:2 * _NBUF]
        gsem = rest[2 * _NBUF:3 * _NBUF]
        osem = rest[3 * _NBUF:]
        wid = lax.axis_index("s") * 2 + lax.axis_index("c")
        i0 = wid * _TOK
        pltpu.sync_copy(idx_hbm.at[wid], idx_v)

        lane = lax.iota(jnp.int32, 16)

        def fire_gather(b, j):
            pltpu.async_copy(table_hbm.at[idx_v.at[j]], g_buf[b], gsem[b])

        def wait_gather(b, j):
            pltpu.make_async_copy(table_hbm.at[idx_v.at[j]], g_buf[b], gsem[b]).wait()

        def out_slice(j):
            return out_hbm.at[pl.ds(j, 1), :, pl.ds(i0, _TOK)]

        def fire_out(b, j):
            pltpu.async_copy(t_buf[b], out_slice(j), osem[b])

        def wait_out(b, j):
            pltpu.make_async_copy(t_buf[b], out_slice(j), osem[b]).wait()

        def transpose(b):
            src = g_buf[b]
            dst = t_buf[b]

            @pl.loop(0, _DIM)
            def _(kk):
                kv = lane * 0 + kk
                for cb in range(_TOK // 16):
                    v = plsc.load_gather(src, [lane + cb * 16, kv])
                    dst[0, kk, pl.ds(cb * 16, 16)] = v

        n_grp = n_cols // _NBUF

        for b in range(_NBUF):
            fire_gather(b, b)
        for b in range(_NBUF):
            wait_gather(b, b)
            transpose(b)
            fire_out(b, b)
            fire_gather(b, _NBUF + b)

        @pl.loop(1, n_grp - 1)
        def _(g):
            j0 = g * _NBUF
            for b in range(_NBUF):
                wait_gather(b, j0 + b)
                wait_out(b, j0 + b - _NBUF)
                transpose(b)
                fire_out(b, j0 + b)
                fire_gather(b, j0 + _NBUF + b)

        j0 = (n_grp - 1) * _NBUF
        for b in range(_NBUF):
            wait_gather(b, j0 + b)
            wait_out(b, j0 + b - _NBUF)
            transpose(b)
            fire_out(b, j0 + b)
        for b in range(_NBUF):
            wait_out(b, j0 + b)

    return k


def kernel(emb_indices, table):
    n_rows, n_cols = emb_indices.shape
    idx_w = emb_indices.T.reshape(n_cols, _NW, _TOK).transpose(1, 0, 2)
    table_p = jnp.pad(table, ((0, 0), (0, _PAD - _DIM)))
    p = _make_gather(n_rows, n_cols)(idx_w, table_p)
    return jnp.transpose(p, (2, 0, 1))
